# baseline (speedup 1.0000x reference)
import functools

import jax
import jax.numpy as jnp
from jax.experimental import pallas as pl
from jax.experimental.pallas import tpu as pltpu

N = 4096
D = 64
BM = 256


def _matmul_block(inp_ref, w_ref, out_ref):
    out_ref[...] = jnp.dot(inp_ref[...], w_ref[...],
                           preferred_element_type=jnp.float32)


@jax.jit
def kernel(inp, weight):
    grid = (N // BM,)
    return pl.pallas_call(
        _matmul_block,
        grid=grid,
        in_specs=[
            pl.BlockSpec((BM, N), lambda i: (i, 0)),
            pl.BlockSpec((N, D), lambda i: (0, 0)),
        ],
        out_specs=pl.BlockSpec((BM, D), lambda i: (i, 0)),
        out_shape=jax.ShapeDtypeStruct((N, D), jnp.float32),
        compiler_params=pltpu.CompilerParams(
            dimension_semantics=("parallel",),
        ),
    )(inp, weight)


# speedup vs baseline: 1.0386x; 1.0386x over previous
import jax
import jax.numpy as jnp
from jax.experimental import pallas as pl
from jax.experimental.pallas import tpu as pltpu

N = 4096
D = 64
BM = 256
NBLK = N // BM
NBUF = 4


def _spmm_kernel(inp_hbm, w_ref, out_ref, bufs, sems):
    def start(i):
        pltpu.make_async_copy(
            inp_hbm.at[pl.ds(i * BM, BM), :],
            bufs.at[i % NBUF],
            sems.at[i % NBUF],
        ).start()

    for i in range(NBUF):
        start(i)
    for i in range(NBLK):
        pltpu.make_async_copy(
            inp_hbm.at[pl.ds(i * BM, BM), :],
            bufs.at[i % NBUF],
            sems.at[i % NBUF],
        ).wait()
        out_ref[pl.ds(i * BM, BM), :] = jnp.dot(
            bufs[i % NBUF], w_ref[...], preferred_element_type=jnp.float32)
        if i + NBUF < NBLK:
            start(i + NBUF)


@jax.jit
def kernel(inp, weight):
    return pl.pallas_call(
        _spmm_kernel,
        in_specs=[
            pl.BlockSpec(memory_space=pltpu.MemorySpace.HBM),
            pl.BlockSpec(memory_space=pltpu.MemorySpace.VMEM),
        ],
        out_specs=pl.BlockSpec(memory_space=pltpu.MemorySpace.VMEM),
        out_shape=jax.ShapeDtypeStruct((N, D), jnp.float32),
        scratch_shapes=[
            pltpu.VMEM((NBUF, BM, N), jnp.float32),
            pltpu.SemaphoreType.DMA((NBUF,)),
        ],
    )(inp, weight)


# speedup vs baseline: 1.0761x; 1.0361x over previous
import jax
import jax.numpy as jnp
from jax.experimental import pallas as pl
from jax.experimental.pallas import tpu as pltpu

N = 4096
D = 64
BM = 256
NBLK = N // BM
NBUF = 4


def _spmm_kernel(inp_hbm, w_ref, out_ref, bufs, sems):
    def start(i):
        pltpu.make_async_copy(
            inp_hbm.at[pl.ds(i * BM, BM), :],
            bufs.at[i % NBUF],
            sems.at[i % NBUF],
        ).start()

    for i in range(NBUF):
        start(i)
    for i in range(NBLK):
        pltpu.make_async_copy(
            inp_hbm.at[pl.ds(i * BM, BM), :],
            bufs.at[i % NBUF],
            sems.at[i % NBUF],
        ).wait()
        out_ref[pl.ds(i * BM, BM), :] = jnp.dot(
            bufs[i % NBUF].astype(jnp.bfloat16),
            w_ref[...].astype(jnp.bfloat16),
            preferred_element_type=jnp.float32)
        if i + NBUF < NBLK:
            start(i + NBUF)


@jax.jit
def kernel(inp, weight):
    return pl.pallas_call(
        _spmm_kernel,
        in_specs=[
            pl.BlockSpec(memory_space=pltpu.MemorySpace.HBM),
            pl.BlockSpec(memory_space=pltpu.MemorySpace.VMEM),
        ],
        out_specs=pl.BlockSpec(memory_space=pltpu.MemorySpace.VMEM),
        out_shape=jax.ShapeDtypeStruct((N, D), jnp.float32),
        scratch_shapes=[
            pltpu.VMEM((NBUF, BM, N), jnp.float32),
            pltpu.SemaphoreType.DMA((NBUF,)),
        ],
    )(inp, weight)
